# baseline (speedup 1.0000x reference)
import jax
import jax.numpy as jnp
from jax import lax
from jax.experimental import pallas as pl
from jax.experimental.pallas import tpu as pltpu
from jax.experimental.pallas import tpu_sc as plsc

N = 10000
E = 320000
D_IN = 256
D_H = 256
D_OUT = 64

NC = 2
NS = 16
CH = 80
DCH = 128

NPAD = 10240
HALF = NPAD // NC
DUMP = HALF
ACC_ROWS = HALF + 128
ROWS_PER_TILE = HALF // NS
ZROWS_PER_TILE = ACC_ROWS // NS

EPAD = 327680
CHUNKS_PER_TILE = EPAD // (NS * CH)
PAIRS_PER_TILE = CHUNKS_PER_TILE // 2
DCHUNKS_PER_TILE = EPAD // (NS * DCH)


def _zero_rows(buf, nrows, width):
  zero16 = jnp.zeros((16,), jnp.float32)
  def zrow(i, _):
    for j in range(width // 16):
      buf[i, pl.ds(j * 16, 16)] = zero16
    return 0
  lax.fori_loop(0, nrows, zrow, 0)


def _zero_acc_slice(zbuf, nrows, acc_s, zbase):
  full, rem = ZROWS_PER_TILE // nrows, ZROWS_PER_TILE % nrows
  for k in range(full):
    pltpu.sync_copy(zbuf, acc_s.at[pl.ds(zbase + k * nrows, nrows)])
  if rem:
    pltpu.sync_copy(zbuf.at[pl.ds(0, rem)],
                    acc_s.at[pl.ds(zbase + full * nrows, rem)])


def _lidx_from_dst(dst_v, lidx_v, node_base, dump, n):
  for j in range(n // 16):
    dd = dst_v[pl.ds(j * 16, 16)]
    inr = (dd >= node_base) & (dd < node_base + HALF)
    lidx_v[pl.ds(j * 16, 16)] = jnp.where(inr, dd - node_base, dump)


def _agg_body(pl_hbm, pr_hbm, src_hbm, dst_hbm, al_hbm, ar_hbm,
              src0, dst0, lidx0, src1, dst1, lidx1,
              rl0, rr0, rl1, rr1, accl_s, accr_s,
              gl0, gr0, gl1, gr1, sl0, sr0, sl1, sr1, isS, isD):
  cid = lax.axis_index("c")
  sid = lax.axis_index("s")
  node_base = cid * HALF
  dump = DUMP + sid

  _zero_rows(rl0, CH, 128)
  zbase = sid * ZROWS_PER_TILE
  _zero_acc_slice(rl0, CH, accl_s, zbase)
  _zero_acc_slice(rl0, CH, accr_s, zbase)
  plsc.subcore_barrier()

  ebase = sid * CHUNKS_PER_TILE * CH

  def load_idx(g, sv, dv):
    eb = ebase + g * CH
    pltpu.async_copy(src_hbm.at[pl.ds(eb, CH)], sv, isS)
    pltpu.async_copy(dst_hbm.at[pl.ds(eb, CH)], dv, isD)

  def wait_idx(sv, dv):
    pltpu.make_async_copy(src_hbm.at[pl.ds(0, CH)], sv, isS).wait()
    pltpu.make_async_copy(dst_hbm.at[pl.ds(0, CH)], dv, isD).wait()

  def fire_gathers(sv, rl, rr, semL, semR):
    pltpu.async_copy(pl_hbm.at[sv], rl, semL)
    pltpu.async_copy(pr_hbm.at[sv], rr, semR)

  def wait_gathers(rl, rr, semL, semR):
    pltpu.make_async_copy(pl_hbm.at[pl.ds(0, CH)], rl, semL).wait()
    pltpu.make_async_copy(pr_hbm.at[pl.ds(0, CH)], rr, semR).wait()

  def fire_scatters(rl, rr, lidx, semL, semR):
    pltpu.async_copy(rl, accl_s.at[lidx], semL, add=True)
    pltpu.async_copy(rr, accr_s.at[lidx], semR, add=True)

  def wait_scatters(rl, rr, semL, semR):
    pltpu.make_async_copy(pl_hbm.at[pl.ds(0, CH)], rl, semL).wait()
    pltpu.make_async_copy(pr_hbm.at[pl.ds(0, CH)], rr, semR).wait()

  load_idx(0, src0, dst0)
  wait_idx(src0, dst0)
  _lidx_from_dst(dst0, lidx0, node_base, dump, CH)
  fire_gathers(src0, rl0, rr0, gl0, gr0)

  def pair(g2, _):
    a = 2 * g2
    wait_gathers(rl0, rr0, gl0, gr0)
    fire_scatters(rl0, rr0, lidx0, sl0, sr0)

    @pl.when(g2 > 0)
    def _():
      wait_scatters(rl1, rr1, sl1, sr1)
    load_idx(a + 1, src1, dst1)
    wait_idx(src1, dst1)
    _lidx_from_dst(dst1, lidx1, node_base, dump, CH)
    fire_gathers(src1, rl1, rr1, gl1, gr1)

    wait_gathers(rl1, rr1, gl1, gr1)
    fire_scatters(rl1, rr1, lidx1, sl1, sr1)

    wait_scatters(rl0, rr0, sl0, sr0)
    @pl.when(g2 < PAIRS_PER_TILE - 1)
    def _():
      load_idx(a + 2, src0, dst0)
      wait_idx(src0, dst0)
      _lidx_from_dst(dst0, lidx0, node_base, dump, CH)
      fire_gathers(src0, rl0, rr0, gl0, gr0)
    return 0
  lax.fori_loop(0, PAIRS_PER_TILE, pair, 0)
  wait_scatters(rl1, rr1, sl1, sr1)
  plsc.subcore_barrier()

  ob = sid * ROWS_PER_TILE
  pltpu.sync_copy(accl_s.at[pl.ds(ob, ROWS_PER_TILE)],
                  al_hbm.at[pl.ds(node_base + ob, ROWS_PER_TILE)])
  pltpu.sync_copy(accr_s.at[pl.ds(ob, ROWS_PER_TILE)],
                  ar_hbm.at[pl.ds(node_base + ob, ROWS_PER_TILE)])


_agg = pl.kernel(
    _agg_body,
    out_type=(jax.ShapeDtypeStruct((NPAD, 128), jnp.float32),
              jax.ShapeDtypeStruct((NPAD, 128), jnp.float32)),
    mesh=plsc.VectorSubcoreMesh(core_axis_name="c", subcore_axis_name="s"),
    scratch_types=[
        pltpu.VMEM((CH,), jnp.int32),
        pltpu.VMEM((CH,), jnp.int32),
        pltpu.VMEM((CH,), jnp.int32),
        pltpu.VMEM((CH,), jnp.int32),
        pltpu.VMEM((CH,), jnp.int32),
        pltpu.VMEM((CH,), jnp.int32),
        pltpu.VMEM((CH, 128), jnp.float32),
        pltpu.VMEM((CH, 128), jnp.float32),
        pltpu.VMEM((CH, 128), jnp.float32),
        pltpu.VMEM((CH, 128), jnp.float32),
        pltpu.VMEM_SHARED((ACC_ROWS, 128), jnp.float32),
        pltpu.VMEM_SHARED((ACC_ROWS, 128), jnp.float32),
        pltpu.SemaphoreType.DMA,
        pltpu.SemaphoreType.DMA,
        pltpu.SemaphoreType.DMA,
        pltpu.SemaphoreType.DMA,
        pltpu.SemaphoreType.DMA,
        pltpu.SemaphoreType.DMA,
        pltpu.SemaphoreType.DMA,
        pltpu.SemaphoreType.DMA,
        pltpu.SemaphoreType.DMA,
        pltpu.SemaphoreType.DMA,
    ])


def _deg_body(dst_hbm, deg_hbm, dst_v, lidx_v, ones_v, dacc_s):
  cid = lax.axis_index("c")
  sid = lax.axis_index("s")
  node_base = cid * HALF
  dump = DUMP + sid

  _zero_rows(ones_v, DCH, 128)
  zbase = sid * ZROWS_PER_TILE
  _zero_acc_slice(ones_v, DCH, dacc_s, zbase)
  ones16 = jnp.ones((16,), jnp.float32)
  def orow(i, _):
    for j in range(8):
      ones_v[i, pl.ds(j * 16, 16)] = ones16
    return 0
  lax.fori_loop(0, DCH, orow, 0)
  plsc.subcore_barrier()

  def chunk(g, _):
    eb = (sid * DCHUNKS_PER_TILE + g) * DCH
    pltpu.sync_copy(dst_hbm.at[pl.ds(eb, DCH)], dst_v)
    _lidx_from_dst(dst_v, lidx_v, node_base, dump, DCH)
    pltpu.sync_copy(ones_v, dacc_s.at[lidx_v], add=True)
    return 0
  lax.fori_loop(0, DCHUNKS_PER_TILE, chunk, 0)
  plsc.subcore_barrier()

  ob = sid * ROWS_PER_TILE
  pltpu.sync_copy(dacc_s.at[pl.ds(ob, ROWS_PER_TILE)],
                  deg_hbm.at[pl.ds(node_base + ob, ROWS_PER_TILE)])


_deg_sc = pl.kernel(
    _deg_body,
    out_type=(jax.ShapeDtypeStruct((NPAD, 128), jnp.float32),),
    mesh=plsc.VectorSubcoreMesh(core_axis_name="c", subcore_axis_name="s"),
    scratch_types=[
        pltpu.VMEM((DCH,), jnp.int32),
        pltpu.VMEM((DCH,), jnp.int32),
        pltpu.VMEM((DCH, 128), jnp.float32),
        pltpu.VMEM_SHARED((ACC_ROWS, 128), jnp.float32),
    ])


BM = 256


def _proj0_body(x_ref, ws_ref, wn_ref, b_ref, s_ref, pl_ref, pr_ref):
  x = x_ref[...]
  s_ref[...] = jnp.dot(x, ws_ref[...],
                       preferred_element_type=jnp.float32) + b_ref[...]
  p = jnp.dot(x, wn_ref[...], preferred_element_type=jnp.float32)
  pl_ref[...] = p[:, :128]
  pr_ref[...] = p[:, 128:]


def _combine_body(s_ref, al_ref, ar_ref, deg_ref, ws_ref, wn_ref, b_ref,
                  so_ref, pl_ref, pr_ref):
  dinv = 1.0 / jnp.maximum(deg_ref[...], 1.0)
  a = jnp.concatenate([al_ref[...], ar_ref[...]], axis=1)
  h = jnp.maximum(s_ref[...] + a * dinv, 0.0)
  so_ref[...] = jnp.dot(h, ws_ref[...],
                        preferred_element_type=jnp.float32) + b_ref[...]
  p = jnp.dot(h, wn_ref[...], preferred_element_type=jnp.float32)
  pl_ref[...] = p[:, :128]
  pr_ref[...] = p[:, 128:]


def _combine_h_body(s_ref, al_ref, ar_ref, deg_ref, ws_ref, b_ref,
                    so_ref, hl_ref, hr_ref):
  dinv = 1.0 / jnp.maximum(deg_ref[...], 1.0)
  a = jnp.concatenate([al_ref[...], ar_ref[...]], axis=1)
  h = jnp.maximum(s_ref[...] + a * dinv, 0.0)
  so_ref[...] = jnp.dot(h, ws_ref[...],
                        preferred_element_type=jnp.float32) + b_ref[...]
  hl_ref[...] = h[:, :128]
  hr_ref[...] = h[:, 128:]


def _final_body(s_ref, al_ref, ar_ref, deg_ref, wn_ref, o_ref):
  dinv = 1.0 / jnp.maximum(deg_ref[...], 1.0)
  a = jnp.concatenate([al_ref[...], ar_ref[...]], axis=1)
  o_ref[...] = s_ref[...] + jnp.dot(a * dinv, wn_ref[...],
                                    preferred_element_type=jnp.float32)


def _row_spec(w):
  return pl.BlockSpec((BM, w), lambda i: (i, 0))


def _full_specs(shapes):
  return [pl.BlockSpec(s, lambda i: (0, 0)) for s in shapes]


def _proj0(x, ws, wn, b):
  return pl.pallas_call(
      _proj0_body,
      grid=(NPAD // BM,),
      in_specs=[_row_spec(D_IN)] + _full_specs([(D_IN, D_H), (D_IN, D_H),
                                                (1, D_H)]),
      out_specs=[_row_spec(D_H), _row_spec(128), _row_spec(128)],
      out_shape=[jax.ShapeDtypeStruct((NPAD, D_H), jnp.float32),
                 jax.ShapeDtypeStruct((NPAD, 128), jnp.float32),
                 jax.ShapeDtypeStruct((NPAD, 128), jnp.float32)],
  )(x, ws, wn, b.reshape(1, -1))


def _combine(s, al, ar, deg, ws, wn, b):
  return pl.pallas_call(
      _combine_body,
      grid=(NPAD // BM,),
      in_specs=[_row_spec(D_H), _row_spec(128), _row_spec(128),
                _row_spec(1)] +
               _full_specs([(D_H, D_H), (D_H, D_H), (1, D_H)]),
      out_specs=[_row_spec(D_H), _row_spec(128), _row_spec(128)],
      out_shape=[jax.ShapeDtypeStruct((NPAD, D_H), jnp.float32),
                 jax.ShapeDtypeStruct((NPAD, 128), jnp.float32),
                 jax.ShapeDtypeStruct((NPAD, 128), jnp.float32)],
  )(s, al, ar, deg, ws, wn, b.reshape(1, -1))


def _combine_h(s, al, ar, deg, ws, b):
  return pl.pallas_call(
      _combine_h_body,
      grid=(NPAD // BM,),
      in_specs=[_row_spec(D_H), _row_spec(128), _row_spec(128),
                _row_spec(1)] +
               _full_specs([(D_H, D_OUT), (1, D_OUT)]),
      out_specs=[_row_spec(D_OUT), _row_spec(128), _row_spec(128)],
      out_shape=[jax.ShapeDtypeStruct((NPAD, D_OUT), jnp.float32),
                 jax.ShapeDtypeStruct((NPAD, 128), jnp.float32),
                 jax.ShapeDtypeStruct((NPAD, 128), jnp.float32)],
  )(s, al, ar, deg, ws, b.reshape(1, -1))


def _final(s, al, ar, deg, wn):
  return pl.pallas_call(
      _final_body,
      grid=(NPAD // BM,),
      in_specs=[_row_spec(D_OUT), _row_spec(128), _row_spec(128),
                _row_spec(1)] + _full_specs([(D_H, D_OUT)]),
      out_specs=_row_spec(D_OUT),
      out_shape=jax.ShapeDtypeStruct((NPAD, D_OUT), jnp.float32),
  )(s, al, ar, deg, wn)


def kernel(features, edge_index, Ws0, Wn0, b0, Ws1, Wn1, b1, Ws2, Wn2, b2):
  x = jnp.zeros((NPAD, D_IN), jnp.float32).at[:N].set(features)
  src = jnp.zeros((EPAD,), jnp.int32).at[:E].set(edge_index[0])
  dst = jnp.full((EPAD,), NPAD, jnp.int32).at[:E].set(edge_index[1])

  deg = _deg_sc(dst)[0][:, :1]
  s0, p0l, p0r = _proj0(x, Ws0, Wn0, b0)
  a0l, a0r = _agg(p0l, p0r, src, dst)
  s1, p1l, p1r = _combine(s0, a0l, a0r, deg, Ws1, Wn1, b1)
  a1l, a1r = _agg(p1l, p1r, src, dst)
  s2, h2l, h2r = _combine_h(s1, a1l, a1r, deg, Ws2, b2)
  a2l, a2r = _agg(h2l, h2r, src, dst)
  out = _final(s2, a2l, a2r, deg, Wn2)
  return out[:N]


# speedup vs baseline: 1.4736x; 1.4736x over previous
import jax
import jax.numpy as jnp
from jax import lax
from jax.experimental import pallas as pl
from jax.experimental.pallas import tpu as pltpu
from jax.experimental.pallas import tpu_sc as plsc

N = 10000
E = 320000
D_IN = 256
D_H = 256
D_OUT = 64

NC = 2
NS = 16
CH = 64
DCH = 128

NPAD = 10240
HALF = NPAD // NC
DUMP = HALF
ACC_ROWS = HALF + 128
ROWS_PER_TILE = HALF // NS
ZROWS_PER_TILE = ACC_ROWS // NS

EPAD = 321536
CHUNKS_PER_TILE = EPAD // (NS * CH)
PAIRS_PER_TILE = CHUNKS_PER_TILE // 2
DCHUNKS_PER_TILE = EPAD // (NS * DCH)


def _zero_rows(buf, nrows, width):
  zero16 = jnp.zeros((16,), jnp.float32)
  def zrow(i, _):
    for j in range(width // 16):
      buf[i, pl.ds(j * 16, 16)] = zero16
    return 0
  lax.fori_loop(0, nrows, zrow, 0)


def _zero_acc_slice(zbuf, nrows, acc_s, zbase):
  full, rem = ZROWS_PER_TILE // nrows, ZROWS_PER_TILE % nrows
  for k in range(full):
    pltpu.sync_copy(zbuf, acc_s.at[pl.ds(zbase + k * nrows, nrows)])
  if rem:
    pltpu.sync_copy(zbuf.at[pl.ds(0, rem)],
                    acc_s.at[pl.ds(zbase + full * nrows, rem)])


def _lidx_from_dst(dst_v, lidx_v, node_base, dump, n):
  for j in range(n // 16):
    dd = dst_v[pl.ds(j * 16, 16)]
    inr = (dd >= node_base) & (dd < node_base + HALF)
    lidx_v[pl.ds(j * 16, 16)] = jnp.where(inr, dd - node_base, dump)


def _agg_body(pl_hbm, pr_hbm, src_hbm, dst_hbm, al_hbm, ar_hbm,
              src0, dst0, lidx0, src1, dst1, lidx1,
              rl0, rr0, rl1, rr1, accl_s, accr_s,
              gl0, gr0, gl1, gr1, sl0, sr0, sl1, sr1, isS, isD):
  cid = lax.axis_index("c")
  sid = lax.axis_index("s")
  node_base = cid * HALF
  dump = DUMP + sid

  _zero_rows(rl0, CH, 128)
  zbase = sid * ZROWS_PER_TILE
  _zero_acc_slice(rl0, CH, accl_s, zbase)
  _zero_acc_slice(rl0, CH, accr_s, zbase)
  plsc.subcore_barrier()

  ebase = sid * CHUNKS_PER_TILE * CH

  def load_idx(g, sv, dv):
    eb = ebase + g * CH
    pltpu.async_copy(src_hbm.at[pl.ds(eb, CH)], sv, isS)
    pltpu.async_copy(dst_hbm.at[pl.ds(eb, CH)], dv, isD)

  def wait_idx(sv, dv):
    pltpu.make_async_copy(src_hbm.at[pl.ds(0, CH)], sv, isS).wait()
    pltpu.make_async_copy(dst_hbm.at[pl.ds(0, CH)], dv, isD).wait()

  def fire_gathers(sv, rl, rr, semL, semR):
    pltpu.async_copy(pl_hbm.at[sv], rl, semL)
    pltpu.async_copy(pr_hbm.at[sv], rr, semR)

  def wait_gathers(rl, rr, semL, semR):
    pltpu.make_async_copy(pl_hbm.at[pl.ds(0, CH)], rl, semL).wait()
    pltpu.make_async_copy(pr_hbm.at[pl.ds(0, CH)], rr, semR).wait()

  def fire_scatters(rl, rr, lidx, semL, semR):
    pltpu.async_copy(rl, accl_s.at[lidx], semL, add=True)
    pltpu.async_copy(rr, accr_s.at[lidx], semR, add=True)

  def wait_scatters(rl, rr, semL, semR):
    pltpu.make_async_copy(pl_hbm.at[pl.ds(0, CH)], rl, semL).wait()
    pltpu.make_async_copy(pr_hbm.at[pl.ds(0, CH)], rr, semR).wait()

  load_idx(0, src0, dst0)
  wait_idx(src0, dst0)
  _lidx_from_dst(dst0, lidx0, node_base, dump, CH)
  fire_gathers(src0, rl0, rr0, gl0, gr0)

  def pair(g2, _):
    a = 2 * g2
    wait_gathers(rl0, rr0, gl0, gr0)
    fire_scatters(rl0, rr0, lidx0, sl0, sr0)

    @pl.when(g2 > 0)
    def _():
      wait_scatters(rl1, rr1, sl1, sr1)
    load_idx(a + 1, src1, dst1)
    wait_idx(src1, dst1)
    _lidx_from_dst(dst1, lidx1, node_base, dump, CH)
    fire_gathers(src1, rl1, rr1, gl1, gr1)

    wait_gathers(rl1, rr1, gl1, gr1)
    fire_scatters(rl1, rr1, lidx1, sl1, sr1)

    wait_scatters(rl0, rr0, sl0, sr0)
    @pl.when(g2 < PAIRS_PER_TILE - 1)
    def _():
      load_idx(a + 2, src0, dst0)
      wait_idx(src0, dst0)
      _lidx_from_dst(dst0, lidx0, node_base, dump, CH)
      fire_gathers(src0, rl0, rr0, gl0, gr0)
    return 0
  lax.fori_loop(0, PAIRS_PER_TILE, pair, 0)
  wait_scatters(rl1, rr1, sl1, sr1)
  plsc.subcore_barrier()

  ob = sid * ROWS_PER_TILE
  pltpu.sync_copy(accl_s.at[pl.ds(ob, ROWS_PER_TILE)],
                  al_hbm.at[pl.ds(node_base + ob, ROWS_PER_TILE)])
  pltpu.sync_copy(accr_s.at[pl.ds(ob, ROWS_PER_TILE)],
                  ar_hbm.at[pl.ds(node_base + ob, ROWS_PER_TILE)])


_agg = pl.kernel(
    _agg_body,
    out_type=(jax.ShapeDtypeStruct((NPAD, 128), jnp.float32),
              jax.ShapeDtypeStruct((NPAD, 128), jnp.float32)),
    mesh=plsc.VectorSubcoreMesh(core_axis_name="c", subcore_axis_name="s"),
    scratch_types=[
        pltpu.VMEM((CH,), jnp.int32),
        pltpu.VMEM((CH,), jnp.int32),
        pltpu.VMEM((CH,), jnp.int32),
        pltpu.VMEM((CH,), jnp.int32),
        pltpu.VMEM((CH,), jnp.int32),
        pltpu.VMEM((CH,), jnp.int32),
        pltpu.VMEM((CH, 128), jnp.float32),
        pltpu.VMEM((CH, 128), jnp.float32),
        pltpu.VMEM((CH, 128), jnp.float32),
        pltpu.VMEM((CH, 128), jnp.float32),
        pltpu.VMEM_SHARED((ACC_ROWS, 128), jnp.float32),
        pltpu.VMEM_SHARED((ACC_ROWS, 128), jnp.float32),
        pltpu.SemaphoreType.DMA,
        pltpu.SemaphoreType.DMA,
        pltpu.SemaphoreType.DMA,
        pltpu.SemaphoreType.DMA,
        pltpu.SemaphoreType.DMA,
        pltpu.SemaphoreType.DMA,
        pltpu.SemaphoreType.DMA,
        pltpu.SemaphoreType.DMA,
        pltpu.SemaphoreType.DMA,
        pltpu.SemaphoreType.DMA,
    ])


def _deg_body(dst_hbm, deg_hbm, dst_v, lidx_v, ones_v, dacc_s):
  cid = lax.axis_index("c")
  sid = lax.axis_index("s")
  node_base = cid * HALF
  dump = DUMP + sid

  _zero_rows(ones_v, DCH, 128)
  zbase = sid * ZROWS_PER_TILE
  _zero_acc_slice(ones_v, DCH, dacc_s, zbase)
  ones16 = jnp.ones((16,), jnp.float32)
  def orow(i, _):
    for j in range(8):
      ones_v[i, pl.ds(j * 16, 16)] = ones16
    return 0
  lax.fori_loop(0, DCH, orow, 0)
  plsc.subcore_barrier()

  def chunk(g, _):
    eb = (sid * DCHUNKS_PER_TILE + g) * DCH
    pltpu.sync_copy(dst_hbm.at[pl.ds(eb, DCH)], dst_v)
    _lidx_from_dst(dst_v, lidx_v, node_base, dump, DCH)
    pltpu.sync_copy(ones_v, dacc_s.at[lidx_v], add=True)
    return 0
  lax.fori_loop(0, DCHUNKS_PER_TILE, chunk, 0)
  plsc.subcore_barrier()

  ob = sid * ROWS_PER_TILE
  pltpu.sync_copy(dacc_s.at[pl.ds(ob, ROWS_PER_TILE)],
                  deg_hbm.at[pl.ds(node_base + ob, ROWS_PER_TILE)])


_deg_sc = pl.kernel(
    _deg_body,
    out_type=(jax.ShapeDtypeStruct((NPAD, 128), jnp.float32),),
    mesh=plsc.VectorSubcoreMesh(core_axis_name="c", subcore_axis_name="s"),
    scratch_types=[
        pltpu.VMEM((DCH,), jnp.int32),
        pltpu.VMEM((DCH,), jnp.int32),
        pltpu.VMEM((DCH, 128), jnp.float32),
        pltpu.VMEM_SHARED((ACC_ROWS, 128), jnp.float32),
    ])


BM = 256


def _proj0_body(x_ref, ws_ref, wn_ref, b_ref, s_ref, pl_ref, pr_ref):
  x = x_ref[...]
  s_ref[...] = jnp.dot(x, ws_ref[...],
                       preferred_element_type=jnp.float32) + b_ref[...]
  p = jnp.dot(x, wn_ref[...], preferred_element_type=jnp.float32)
  pl_ref[...] = p[:, :128]
  pr_ref[...] = p[:, 128:]


def _combine_body(s_ref, al_ref, ar_ref, deg_ref, ws_ref, wn_ref, b_ref,
                  so_ref, pl_ref, pr_ref):
  dinv = 1.0 / jnp.maximum(deg_ref[...], 1.0)
  a = jnp.concatenate([al_ref[...], ar_ref[...]], axis=1)
  h = jnp.maximum(s_ref[...] + a * dinv, 0.0)
  so_ref[...] = jnp.dot(h, ws_ref[...],
                        preferred_element_type=jnp.float32) + b_ref[...]
  p = jnp.dot(h, wn_ref[...], preferred_element_type=jnp.float32)
  pl_ref[...] = p[:, :128]
  pr_ref[...] = p[:, 128:]


def _combine_h_body(s_ref, al_ref, ar_ref, deg_ref, ws_ref, b_ref,
                    so_ref, hl_ref, hr_ref):
  dinv = 1.0 / jnp.maximum(deg_ref[...], 1.0)
  a = jnp.concatenate([al_ref[...], ar_ref[...]], axis=1)
  h = jnp.maximum(s_ref[...] + a * dinv, 0.0)
  so_ref[...] = jnp.dot(h, ws_ref[...],
                        preferred_element_type=jnp.float32) + b_ref[...]
  hl_ref[...] = h[:, :128]
  hr_ref[...] = h[:, 128:]


def _final_body(s_ref, al_ref, ar_ref, deg_ref, wn_ref, o_ref):
  dinv = 1.0 / jnp.maximum(deg_ref[...], 1.0)
  a = jnp.concatenate([al_ref[...], ar_ref[...]], axis=1)
  o_ref[...] = s_ref[...] + jnp.dot(a * dinv, wn_ref[...],
                                    preferred_element_type=jnp.float32)


def _row_spec(w):
  return pl.BlockSpec((BM, w), lambda i: (i, 0))


def _full_specs(shapes):
  return [pl.BlockSpec(s, lambda i: (0, 0)) for s in shapes]


def _proj0(x, ws, wn, b):
  return pl.pallas_call(
      _proj0_body,
      grid=(NPAD // BM,),
      in_specs=[_row_spec(D_IN)] + _full_specs([(D_IN, D_H), (D_IN, D_H),
                                                (1, D_H)]),
      out_specs=[_row_spec(D_H), _row_spec(128), _row_spec(128)],
      out_shape=[jax.ShapeDtypeStruct((NPAD, D_H), jnp.float32),
                 jax.ShapeDtypeStruct((NPAD, 128), jnp.float32),
                 jax.ShapeDtypeStruct((NPAD, 128), jnp.float32)],
  )(x, ws, wn, b.reshape(1, -1))


def _combine(s, al, ar, deg, ws, wn, b):
  return pl.pallas_call(
      _combine_body,
      grid=(NPAD // BM,),
      in_specs=[_row_spec(D_H), _row_spec(128), _row_spec(128),
                _row_spec(1)] +
               _full_specs([(D_H, D_H), (D_H, D_H), (1, D_H)]),
      out_specs=[_row_spec(D_H), _row_spec(128), _row_spec(128)],
      out_shape=[jax.ShapeDtypeStruct((NPAD, D_H), jnp.float32),
                 jax.ShapeDtypeStruct((NPAD, 128), jnp.float32),
                 jax.ShapeDtypeStruct((NPAD, 128), jnp.float32)],
  )(s, al, ar, deg, ws, wn, b.reshape(1, -1))


def _combine_h(s, al, ar, deg, ws, b):
  return pl.pallas_call(
      _combine_h_body,
      grid=(NPAD // BM,),
      in_specs=[_row_spec(D_H), _row_spec(128), _row_spec(128),
                _row_spec(1)] +
               _full_specs([(D_H, D_OUT), (1, D_OUT)]),
      out_specs=[_row_spec(D_OUT), _row_spec(128), _row_spec(128)],
      out_shape=[jax.ShapeDtypeStruct((NPAD, D_OUT), jnp.float32),
                 jax.ShapeDtypeStruct((NPAD, 128), jnp.float32),
                 jax.ShapeDtypeStruct((NPAD, 128), jnp.float32)],
  )(s, al, ar, deg, ws, b.reshape(1, -1))


def _final(s, al, ar, deg, wn):
  return pl.pallas_call(
      _final_body,
      grid=(NPAD // BM,),
      in_specs=[_row_spec(D_OUT), _row_spec(128), _row_spec(128),
                _row_spec(1)] + _full_specs([(D_H, D_OUT)]),
      out_specs=_row_spec(D_OUT),
      out_shape=jax.ShapeDtypeStruct((NPAD, D_OUT), jnp.float32),
  )(s, al, ar, deg, wn)


def kernel(features, edge_index, Ws0, Wn0, b0, Ws1, Wn1, b1, Ws2, Wn2, b2):
  x = jnp.zeros((NPAD, D_IN), jnp.float32).at[:N].set(features)
  src = jnp.zeros((EPAD,), jnp.int32).at[:E].set(edge_index[0])
  dst = jnp.full((EPAD,), NPAD, jnp.int32).at[:E].set(edge_index[1])

  deg = _deg_sc(dst)[0][:, :1]
  s0, p0l, p0r = _proj0(x, Ws0, Wn0, b0)
  a0l, a0r = _agg(p0l, p0r, src, dst)
  s1, p1l, p1r = _combine(s0, a0l, a0r, deg, Ws1, Wn1, b1)
  a1l, a1r = _agg(p1l, p1r, src, dst)
  s2, h2l, h2r = _combine_h(s1, a1l, a1r, deg, Ws2, b2)
  a2l, a2r = _agg(h2l, h2r, src, dst)
  out = _final(s2, a2l, a2r, deg, Wn2)
  return out[:N]


# speedup vs baseline: 1.4980x; 1.0165x over previous
import jax
import jax.numpy as jnp
from jax import lax
from jax.experimental import pallas as pl
from jax.experimental.pallas import tpu as pltpu
from jax.experimental.pallas import tpu_sc as plsc

N = 10000
E = 320000
D_IN = 256
D_H = 256
D_OUT = 64

NC = 2
NS = 16
CH = 64
DCH = 64

NPAD = 10240
HALF = NPAD // NC
DUMP = HALF
ACC_ROWS = HALF + 128
ROWS_PER_TILE = HALF // NS
ZROWS_PER_TILE = ACC_ROWS // NS

EPAD = 321536
CHUNKS_PER_TILE = EPAD // (NS * CH)
PAIRS_PER_TILE = CHUNKS_PER_TILE // 2
DCHUNKS_PER_TILE = EPAD // (NS * DCH)
DPAIRS_PER_TILE = DCHUNKS_PER_TILE // 2


def _zero_rows(buf, nrows, width):
  zero16 = jnp.zeros((16,), jnp.float32)
  def zrow(i, _):
    for j in range(width // 16):
      buf[i, pl.ds(j * 16, 16)] = zero16
    return 0
  lax.fori_loop(0, nrows, zrow, 0)


def _zero_acc_slice(zbuf, nrows, acc_s, zbase):
  full, rem = ZROWS_PER_TILE // nrows, ZROWS_PER_TILE % nrows
  for k in range(full):
    pltpu.sync_copy(zbuf, acc_s.at[pl.ds(zbase + k * nrows, nrows)])
  if rem:
    pltpu.sync_copy(zbuf.at[pl.ds(0, rem)],
                    acc_s.at[pl.ds(zbase + full * nrows, rem)])


def _lidx_from_dst(dst_v, lidx_v, node_base, dump, n):
  for j in range(n // 16):
    dd = dst_v[pl.ds(j * 16, 16)]
    inr = (dd >= node_base) & (dd < node_base + HALF)
    lidx_v[pl.ds(j * 16, 16)] = jnp.where(inr, dd - node_base, dump)


def _agg_body(pl_hbm, pr_hbm, src_hbm, dst_hbm, al_hbm, ar_hbm,
              src0, dst0, lidx0, src1, dst1, lidx1,
              rl0, rr0, rl1, rr1, accl_s, accr_s,
              gl0, gr0, gl1, gr1, sl0, sr0, sl1, sr1, isS, isD):
  cid = lax.axis_index("c")
  sid = lax.axis_index("s")
  node_base = cid * HALF
  dump = DUMP + sid

  _zero_rows(rl0, CH, 128)
  zbase = sid * ZROWS_PER_TILE
  _zero_acc_slice(rl0, CH, accl_s, zbase)
  _zero_acc_slice(rl0, CH, accr_s, zbase)
  plsc.subcore_barrier()

  ebase = sid * CHUNKS_PER_TILE * CH

  def load_idx(g, sv, dv):
    eb = ebase + g * CH
    pltpu.async_copy(src_hbm.at[pl.ds(eb, CH)], sv, isS)
    pltpu.async_copy(dst_hbm.at[pl.ds(eb, CH)], dv, isD)

  def wait_idx(sv, dv):
    pltpu.make_async_copy(src_hbm.at[pl.ds(0, CH)], sv, isS).wait()
    pltpu.make_async_copy(dst_hbm.at[pl.ds(0, CH)], dv, isD).wait()

  def fire_gathers(sv, rl, rr, semL, semR):
    pltpu.async_copy(pl_hbm.at[sv], rl, semL)
    pltpu.async_copy(pr_hbm.at[sv], rr, semR)

  def wait_gathers(rl, rr, semL, semR):
    pltpu.make_async_copy(pl_hbm.at[pl.ds(0, CH)], rl, semL).wait()
    pltpu.make_async_copy(pr_hbm.at[pl.ds(0, CH)], rr, semR).wait()

  def fire_scatters(rl, rr, lidx, semL, semR):
    pltpu.async_copy(rl, accl_s.at[lidx], semL, add=True)
    pltpu.async_copy(rr, accr_s.at[lidx], semR, add=True)

  def wait_scatters(rl, rr, semL, semR):
    pltpu.make_async_copy(pl_hbm.at[pl.ds(0, CH)], rl, semL).wait()
    pltpu.make_async_copy(pr_hbm.at[pl.ds(0, CH)], rr, semR).wait()

  load_idx(0, src0, dst0)
  wait_idx(src0, dst0)
  _lidx_from_dst(dst0, lidx0, node_base, dump, CH)
  fire_gathers(src0, rl0, rr0, gl0, gr0)

  def pair(g2, _):
    a = 2 * g2
    wait_gathers(rl0, rr0, gl0, gr0)
    fire_scatters(rl0, rr0, lidx0, sl0, sr0)

    @pl.when(g2 > 0)
    def _():
      wait_scatters(rl1, rr1, sl1, sr1)
    load_idx(a + 1, src1, dst1)
    wait_idx(src1, dst1)
    _lidx_from_dst(dst1, lidx1, node_base, dump, CH)
    fire_gathers(src1, rl1, rr1, gl1, gr1)

    wait_gathers(rl1, rr1, gl1, gr1)
    fire_scatters(rl1, rr1, lidx1, sl1, sr1)

    wait_scatters(rl0, rr0, sl0, sr0)
    @pl.when(g2 < PAIRS_PER_TILE - 1)
    def _():
      load_idx(a + 2, src0, dst0)
      wait_idx(src0, dst0)
      _lidx_from_dst(dst0, lidx0, node_base, dump, CH)
      fire_gathers(src0, rl0, rr0, gl0, gr0)
    return 0
  lax.fori_loop(0, PAIRS_PER_TILE, pair, 0)
  wait_scatters(rl1, rr1, sl1, sr1)
  plsc.subcore_barrier()

  ob = sid * ROWS_PER_TILE
  pltpu.sync_copy(accl_s.at[pl.ds(ob, ROWS_PER_TILE)],
                  al_hbm.at[pl.ds(node_base + ob, ROWS_PER_TILE)])
  pltpu.sync_copy(accr_s.at[pl.ds(ob, ROWS_PER_TILE)],
                  ar_hbm.at[pl.ds(node_base + ob, ROWS_PER_TILE)])


_agg = pl.kernel(
    _agg_body,
    out_type=(jax.ShapeDtypeStruct((NPAD, 128), jnp.float32),
              jax.ShapeDtypeStruct((NPAD, 128), jnp.float32)),
    mesh=plsc.VectorSubcoreMesh(core_axis_name="c", subcore_axis_name="s"),
    scratch_types=[
        pltpu.VMEM((CH,), jnp.int32),
        pltpu.VMEM((CH,), jnp.int32),
        pltpu.VMEM((CH,), jnp.int32),
        pltpu.VMEM((CH,), jnp.int32),
        pltpu.VMEM((CH,), jnp.int32),
        pltpu.VMEM((CH,), jnp.int32),
        pltpu.VMEM((CH, 128), jnp.float32),
        pltpu.VMEM((CH, 128), jnp.float32),
        pltpu.VMEM((CH, 128), jnp.float32),
        pltpu.VMEM((CH, 128), jnp.float32),
        pltpu.VMEM_SHARED((ACC_ROWS, 128), jnp.float32),
        pltpu.VMEM_SHARED((ACC_ROWS, 128), jnp.float32),
        pltpu.SemaphoreType.DMA,
        pltpu.SemaphoreType.DMA,
        pltpu.SemaphoreType.DMA,
        pltpu.SemaphoreType.DMA,
        pltpu.SemaphoreType.DMA,
        pltpu.SemaphoreType.DMA,
        pltpu.SemaphoreType.DMA,
        pltpu.SemaphoreType.DMA,
        pltpu.SemaphoreType.DMA,
        pltpu.SemaphoreType.DMA,
    ])


def _deg_body(dst_hbm, deg_hbm, dst0, dst1, lidx0, lidx1, ones_v, dacc_s,
              isD, ss0, ss1):
  cid = lax.axis_index("c")
  sid = lax.axis_index("s")
  node_base = cid * HALF
  dump = DUMP + sid

  _zero_rows(ones_v, DCH, 128)
  zbase = sid * ZROWS_PER_TILE
  _zero_acc_slice(ones_v, DCH, dacc_s, zbase)
  ones16 = jnp.ones((16,), jnp.float32)
  def orow(i, _):
    for j in range(8):
      ones_v[i, pl.ds(j * 16, 16)] = ones16
    return 0
  lax.fori_loop(0, DCH, orow, 0)
  plsc.subcore_barrier()

  ebase = sid * DCHUNKS_PER_TILE * DCH

  def load_idx(g, dv):
    pltpu.async_copy(dst_hbm.at[pl.ds(ebase + g * DCH, DCH)], dv, isD)

  def wait_idx(dv):
    pltpu.make_async_copy(dst_hbm.at[pl.ds(0, DCH)], dv, isD).wait()

  def fire_scatter(lidx, sem):
    pltpu.async_copy(ones_v, dacc_s.at[lidx], sem, add=True)

  def wait_scatter(sem):
    pltpu.make_async_copy(deg_hbm.at[pl.ds(0, DCH)], ones_v, sem).wait()

  load_idx(0, dst0)
  wait_idx(dst0)
  _lidx_from_dst(dst0, lidx0, node_base, dump, DCH)

  def pair(g2, _):
    a = 2 * g2
    fire_scatter(lidx0, ss0)
    @pl.when(g2 > 0)
    def _():
      wait_scatter(ss1)
    load_idx(a + 1, dst1)
    wait_idx(dst1)
    _lidx_from_dst(dst1, lidx1, node_base, dump, DCH)
    fire_scatter(lidx1, ss1)
    wait_scatter(ss0)
    @pl.when(g2 < DPAIRS_PER_TILE - 1)
    def _():
      load_idx(a + 2, dst0)
      wait_idx(dst0)
      _lidx_from_dst(dst0, lidx0, node_base, dump, DCH)
    return 0
  lax.fori_loop(0, DPAIRS_PER_TILE, pair, 0)
  wait_scatter(ss1)
  plsc.subcore_barrier()

  ob = sid * ROWS_PER_TILE
  pltpu.sync_copy(dacc_s.at[pl.ds(ob, ROWS_PER_TILE)],
                  deg_hbm.at[pl.ds(node_base + ob, ROWS_PER_TILE)])


_deg_sc = pl.kernel(
    _deg_body,
    out_type=(jax.ShapeDtypeStruct((NPAD, 128), jnp.float32),),
    mesh=plsc.VectorSubcoreMesh(core_axis_name="c", subcore_axis_name="s"),
    scratch_types=[
        pltpu.VMEM((DCH,), jnp.int32),
        pltpu.VMEM((DCH,), jnp.int32),
        pltpu.VMEM((DCH,), jnp.int32),
        pltpu.VMEM((DCH,), jnp.int32),
        pltpu.VMEM((DCH, 128), jnp.float32),
        pltpu.VMEM_SHARED((ACC_ROWS, 128), jnp.float32),
        pltpu.SemaphoreType.DMA,
        pltpu.SemaphoreType.DMA,
        pltpu.SemaphoreType.DMA,
    ])


BM = 256


def _proj0_body(x_ref, ws_ref, wn_ref, b_ref, s_ref, pl_ref, pr_ref):
  x = x_ref[...]
  s_ref[...] = jnp.dot(x, ws_ref[...],
                       preferred_element_type=jnp.float32) + b_ref[...]
  p = jnp.dot(x, wn_ref[...], preferred_element_type=jnp.float32)
  pl_ref[...] = p[:, :128]
  pr_ref[...] = p[:, 128:]


def _combine_body(s_ref, al_ref, ar_ref, deg_ref, ws_ref, wn_ref, b_ref,
                  so_ref, pl_ref, pr_ref):
  dinv = 1.0 / jnp.maximum(deg_ref[...], 1.0)
  a = jnp.concatenate([al_ref[...], ar_ref[...]], axis=1)
  h = jnp.maximum(s_ref[...] + a * dinv, 0.0)
  so_ref[...] = jnp.dot(h, ws_ref[...],
                        preferred_element_type=jnp.float32) + b_ref[...]
  p = jnp.dot(h, wn_ref[...], preferred_element_type=jnp.float32)
  pl_ref[...] = p[:, :128]
  pr_ref[...] = p[:, 128:]


def _combine_h_body(s_ref, al_ref, ar_ref, deg_ref, ws_ref, b_ref,
                    so_ref, hl_ref, hr_ref):
  dinv = 1.0 / jnp.maximum(deg_ref[...], 1.0)
  a = jnp.concatenate([al_ref[...], ar_ref[...]], axis=1)
  h = jnp.maximum(s_ref[...] + a * dinv, 0.0)
  so_ref[...] = jnp.dot(h, ws_ref[...],
                        preferred_element_type=jnp.float32) + b_ref[...]
  hl_ref[...] = h[:, :128]
  hr_ref[...] = h[:, 128:]


def _final_body(s_ref, al_ref, ar_ref, deg_ref, wn_ref, o_ref):
  dinv = 1.0 / jnp.maximum(deg_ref[...], 1.0)
  a = jnp.concatenate([al_ref[...], ar_ref[...]], axis=1)
  o_ref[...] = s_ref[...] + jnp.dot(a * dinv, wn_ref[...],
                                    preferred_element_type=jnp.float32)


def _row_spec(w):
  return pl.BlockSpec((BM, w), lambda i: (i, 0))


def _full_specs(shapes):
  return [pl.BlockSpec(s, lambda i: (0, 0)) for s in shapes]


def _proj0(x, ws, wn, b):
  return pl.pallas_call(
      _proj0_body,
      grid=(NPAD // BM,),
      in_specs=[_row_spec(D_IN)] + _full_specs([(D_IN, D_H), (D_IN, D_H),
                                                (1, D_H)]),
      out_specs=[_row_spec(D_H), _row_spec(128), _row_spec(128)],
      out_shape=[jax.ShapeDtypeStruct((NPAD, D_H), jnp.float32),
                 jax.ShapeDtypeStruct((NPAD, 128), jnp.float32),
                 jax.ShapeDtypeStruct((NPAD, 128), jnp.float32)],
  )(x, ws, wn, b.reshape(1, -1))


def _combine(s, al, ar, deg, ws, wn, b):
  return pl.pallas_call(
      _combine_body,
      grid=(NPAD // BM,),
      in_specs=[_row_spec(D_H), _row_spec(128), _row_spec(128),
                _row_spec(1)] +
               _full_specs([(D_H, D_H), (D_H, D_H), (1, D_H)]),
      out_specs=[_row_spec(D_H), _row_spec(128), _row_spec(128)],
      out_shape=[jax.ShapeDtypeStruct((NPAD, D_H), jnp.float32),
                 jax.ShapeDtypeStruct((NPAD, 128), jnp.float32),
                 jax.ShapeDtypeStruct((NPAD, 128), jnp.float32)],
  )(s, al, ar, deg, ws, wn, b.reshape(1, -1))


def _combine_h(s, al, ar, deg, ws, b):
  return pl.pallas_call(
      _combine_h_body,
      grid=(NPAD // BM,),
      in_specs=[_row_spec(D_H), _row_spec(128), _row_spec(128),
                _row_spec(1)] +
               _full_specs([(D_H, D_OUT), (1, D_OUT)]),
      out_specs=[_row_spec(D_OUT), _row_spec(128), _row_spec(128)],
      out_shape=[jax.ShapeDtypeStruct((NPAD, D_OUT), jnp.float32),
                 jax.ShapeDtypeStruct((NPAD, 128), jnp.float32),
                 jax.ShapeDtypeStruct((NPAD, 128), jnp.float32)],
  )(s, al, ar, deg, ws, b.reshape(1, -1))


def _final(s, al, ar, deg, wn):
  return pl.pallas_call(
      _final_body,
      grid=(NPAD // BM,),
      in_specs=[_row_spec(D_OUT), _row_spec(128), _row_spec(128),
                _row_spec(1)] + _full_specs([(D_H, D_OUT)]),
      out_specs=_row_spec(D_OUT),
      out_shape=jax.ShapeDtypeStruct((NPAD, D_OUT), jnp.float32),
  )(s, al, ar, deg, wn)


def kernel(features, edge_index, Ws0, Wn0, b0, Ws1, Wn1, b1, Ws2, Wn2, b2):
  x = jnp.zeros((NPAD, D_IN), jnp.float32).at[:N].set(features)
  src = jnp.zeros((EPAD,), jnp.int32).at[:E].set(edge_index[0])
  dst = jnp.full((EPAD,), NPAD, jnp.int32).at[:E].set(edge_index[1])

  deg = _deg_sc(dst)[0][:, :1]
  s0, p0l, p0r = _proj0(x, Ws0, Wn0, b0)
  a0l, a0r = _agg(p0l, p0r, src, dst)
  s1, p1l, p1r = _combine(s0, a0l, a0r, deg, Ws1, Wn1, b1)
  a1l, a1r = _agg(p1l, p1r, src, dst)
  s2, h2l, h2r = _combine_h(s1, a1l, a1r, deg, Ws2, b2)
  a2l, a2r = _agg(h2l, h2r, src, dst)
  out = _final(s2, a2l, a2r, deg, Wn2)
  return out[:N]
